# Initial kernel scaffold; baseline (speedup 1.0000x reference)
import jax
import jax.numpy as jnp
from jax.experimental import pallas as pl


def kernel(labels, train, table):
    raise NotImplementedError("write your pallas kernel here")


# speedup vs baseline: 2.4537x; 2.4537x over previous
import functools

import jax
import jax.numpy as jnp
from jax import lax
from jax.experimental import pallas as pl
from jax.experimental.pallas import tpu as pltpu
from jax.experimental.pallas import tpu_sc as plsc

NUM_CLASSES = 1000
HIDDEN_SIZE = 128
DROPOUT_PROB = 0.1
BATCH = 16384

_NC = 2
_NS = 16
_NW = _NC * _NS
_B_PER_W = BATCH // _NW
_CHUNK = 128
_N_CHUNKS = _B_PER_W // _CHUNK


def _embed_body(table_hbm, idx_hbm, out_hbm, idx_v, rows_v, sem):
    wid = lax.axis_index("s") * _NC + lax.axis_index("c")
    base = wid * _B_PER_W
    pltpu.sync_copy(idx_hbm.at[pl.ds(base, _B_PER_W)], idx_v)
    copies = []
    for c in range(_N_CHUNKS):
        copies.append(
            pltpu.async_copy(
                table_hbm.at[idx_v.at[pl.ds(c * _CHUNK, _CHUNK)]],
                rows_v.at[pl.ds(c * _CHUNK, _CHUNK)],
                sem,
            )
        )
    for cp in copies:
        cp.wait()
    pltpu.sync_copy(rows_v, out_hbm.at[pl.ds(base, _B_PER_W)])


@jax.jit
def _embed(table, idx):
    mesh = plsc.VectorSubcoreMesh(core_axis_name="c", subcore_axis_name="s")
    return pl.kernel(
        _embed_body,
        mesh=mesh,
        out_type=jax.ShapeDtypeStruct((BATCH, HIDDEN_SIZE), jnp.float32),
        scratch_types=[
            pltpu.VMEM((_B_PER_W,), jnp.int32),
            pltpu.VMEM((_B_PER_W, HIDDEN_SIZE), jnp.float32),
            pltpu.SemaphoreType.DMA,
        ],
    )(table, idx)


def kernel(labels, train, table):
    use_drop = jnp.logical_and(jnp.asarray(train) != 0, DROPOUT_PROB > 0.0)
    drop_key = jax.random.key(1)
    drop_ids = jax.random.uniform(drop_key, (labels.shape[0],)) < DROPOUT_PROB
    idx = jnp.where(jnp.logical_and(use_drop, drop_ids), NUM_CLASSES, labels)
    return _embed(table, idx.astype(jnp.int32))
